# Initial kernel scaffold; baseline (speedup 1.0000x reference)
import functools

import jax
import jax.numpy as jnp
from jax import lax
from jax.experimental import pallas as pl
from jax.experimental.pallas import tpu as pltpu
from jax.experimental.pallas import tpu_sc as plsc

N = 10000
D = 128
H = 128
E = 320000

NC = 2
NS = 16
NW = NC * NS
CHUNK = 128
T_PER_W = 79
E_PAD = NW * T_PER_W * CHUNK
NP = N + 16
RPS_ACC = NP // NS
RPS_OUT = N // NS

_mesh = plsc.VectorSubcoreMesh(core_axis_name="c", subcore_axis_name="s")


def _make_sc_aggregate(with_counts: bool):
    out_type = [jax.ShapeDtypeStruct((NC, N, D), jnp.float32)]
    scratch = [
        pltpu.VMEM((T_PER_W, CHUNK), jnp.int32),
        pltpu.VMEM((T_PER_W, CHUNK), jnp.int32),
        pltpu.VMEM((CHUNK, D), jnp.float32),
        pltpu.VMEM_SHARED((NP, D), jnp.float32),
        pltpu.SemaphoreType.DMA,
    ]
    if with_counts:
        out_type.append(jax.ShapeDtypeStruct((NC, N, 16), jnp.float32))
        scratch += [
            pltpu.VMEM((CHUNK, 16), jnp.float32),
            pltpu.VMEM_SHARED((NP, 16), jnp.float32),
        ]

    @functools.partial(pl.kernel, out_type=out_type, mesh=_mesh,
                       scratch_types=scratch)
    def sc_agg(x_hbm, src_hbm, dst_hbm, zf_hbm, zc_hbm, parts_hbm, *rest):
        if with_counts:
            cnts_hbm, src_v, dst_v, rows_v, acc, sem, ones_v, cacc = rest
        else:
            src_v, dst_v, rows_v, acc, sem = rest
        cid = lax.axis_index("c")
        sid = lax.axis_index("s")
        wid = sid * NC + cid

        pltpu.sync_copy(zf_hbm.at[pl.ds(sid * RPS_ACC, RPS_ACC)],
                        acc.at[pl.ds(sid * RPS_ACC, RPS_ACC)])
        if with_counts:
            pltpu.sync_copy(zc_hbm.at[pl.ds(sid * RPS_ACC, RPS_ACC)],
                            cacc.at[pl.ds(sid * RPS_ACC, RPS_ACC)])
            @pl.loop(0, CHUNK)
            def _(i):
                ones_v[i, :] = jnp.full((16,), 1.0, jnp.float32)

        pltpu.sync_copy(src_hbm.at[pl.ds(wid * T_PER_W, T_PER_W)], src_v)
        pltpu.sync_copy(dst_hbm.at[pl.ds(wid * T_PER_W, T_PER_W)], dst_v)
        plsc.subcore_barrier()

        @pl.loop(0, T_PER_W)
        def _(t):
            pltpu.async_copy(x_hbm.at[src_v.at[t]], rows_v, sem).wait()
            pltpu.sync_copy(rows_v, acc.at[dst_v.at[t]], add=True)
            if with_counts:
                pltpu.sync_copy(ones_v, cacc.at[dst_v.at[t]], add=True)

        plsc.subcore_barrier()
        pltpu.sync_copy(acc.at[pl.ds(sid * RPS_OUT, RPS_OUT)],
                        parts_hbm.at[cid, pl.ds(sid * RPS_OUT, RPS_OUT)])
        if with_counts:
            pltpu.sync_copy(cacc.at[pl.ds(sid * RPS_OUT, RPS_OUT)],
                            cnts_hbm.at[cid, pl.ds(sid * RPS_OUT, RPS_OUT)])

    return sc_agg


_sc_agg_counts = _make_sc_aggregate(True)
_sc_agg = _make_sc_aggregate(False)

BR = 1000
_PREC = lax.Precision.HIGHEST


def _dense1_body(parts_ref, cnts_ref, x_ref, wl_ref, b_ref, wr_ref, o_ref):
    p = parts_ref[0] + parts_ref[1]
    c = cnts_ref[0, :, 0:1] + cnts_ref[1, :, 0:1]
    mean = p / jnp.maximum(c, 1.0)
    h = (jnp.dot(mean, wl_ref[...], precision=_PREC,
                 preferred_element_type=jnp.float32)
         + b_ref[...]
         + jnp.dot(x_ref[...], wr_ref[...], precision=_PREC,
                   preferred_element_type=jnp.float32))
    o_ref[...] = jnp.maximum(h, 0.0)


def _dense2_body(parts_ref, cnts_ref, h1_ref, x_ref, wl_ref, b_ref, wr_ref,
                 wo_ref, bo_ref, o_ref):
    p = parts_ref[0] + parts_ref[1]
    c = cnts_ref[0, :, 0:1] + cnts_ref[1, :, 0:1]
    mean = p / jnp.maximum(c, 1.0)
    h2 = (jnp.dot(mean, wl_ref[...], precision=_PREC,
                  preferred_element_type=jnp.float32)
          + b_ref[...]
          + jnp.dot(h1_ref[...], wr_ref[...], precision=_PREC,
                    preferred_element_type=jnp.float32))
    h2 = jnp.maximum(h2, 0.0)
    out = x_ref[...] + jnp.dot(h2, wo_ref[...], precision=_PREC,
                               preferred_element_type=jnp.float32) + bo_ref[...]
    o_ref[...] = jnp.maximum(out, 0.0)


_spec_parts = pl.BlockSpec((NC, BR, D), lambda i: (0, i, 0))
_spec_cnts = pl.BlockSpec((NC, BR, 16), lambda i: (0, i, 0))
_spec_rows = pl.BlockSpec((BR, D), lambda i: (i, 0))
_spec_w = pl.BlockSpec((D, H), lambda i: (0, 0))
_spec_b = pl.BlockSpec((1, H), lambda i: (0, 0))

_dense1 = pl.pallas_call(
    _dense1_body,
    grid=(N // BR,),
    in_specs=[_spec_parts, _spec_cnts, _spec_rows, _spec_w, _spec_b, _spec_w],
    out_specs=_spec_rows,
    out_shape=jax.ShapeDtypeStruct((N, H), jnp.float32),
)

_dense2 = pl.pallas_call(
    _dense2_body,
    grid=(N // BR,),
    in_specs=[_spec_parts, _spec_cnts, _spec_rows, _spec_rows, _spec_w,
              _spec_b, _spec_w, _spec_w, _spec_b],
    out_specs=_spec_rows,
    out_shape=jax.ShapeDtypeStruct((N, D), jnp.float32),
)


@jax.jit
def kernel(x, edge_index, W1l, b1l, W1r, W2l, b2l, W2r, Wout, bout):
    src = edge_index[0].astype(jnp.int32)
    dst = edge_index[1].astype(jnp.int32)
    pad = E_PAD - E
    src_p = jnp.concatenate([src, jnp.zeros((pad,), jnp.int32)])
    dst_p = jnp.concatenate([dst, jnp.full((pad,), N, jnp.int32)])
    src_r = src_p.reshape(NW * T_PER_W, CHUNK)
    dst_r = dst_p.reshape(NW * T_PER_W, CHUNK)
    zf = jnp.zeros((NP, D), jnp.float32)
    zc = jnp.zeros((NP, 16), jnp.float32)

    parts1, cnts = _sc_agg_counts(x, src_r, dst_r, zf, zc)
    h1 = _dense1(parts1, cnts, x, W1l.T, b1l.reshape(1, H), W1r.T)
    parts2 = _sc_agg(h1, src_r, dst_r, zf, zc)
    out = _dense2(parts2, cnts, h1, x, W2l.T, b2l.reshape(1, H), W2r.T,
                  Wout.T, bout.reshape(1, D))
    return out


# speedup vs baseline: 7.7747x; 7.7747x over previous
import functools

import jax
import jax.numpy as jnp
from jax import lax
from jax.experimental import pallas as pl
from jax.experimental.pallas import tpu as pltpu
from jax.experimental.pallas import tpu_sc as plsc

N = 10000
D = 128
H = 128
E = 320000

NC = 2
NS = 16
NW = NC * NS
CHUNK = 128
T_PER_W = 80
KI = 16
E_PAD = NW * T_PER_W * CHUNK
NP = 10112
RPS_ACC = NP // NS
RPS_OUT = 624
TAIL = N - NS * RPS_OUT

_mesh = plsc.VectorSubcoreMesh(core_axis_name="c", subcore_axis_name="s",
                               num_cores=NC, num_subcores=NS)


@functools.partial(
    pl.kernel,
    out_type=jax.ShapeDtypeStruct((NC, N, D), jnp.float32),
    mesh=_mesh,
    scratch_types=[
        pltpu.VMEM((KI, CHUNK), jnp.int32),
        pltpu.VMEM((CHUNK,), jnp.int32),
        pltpu.VMEM((CHUNK,), jnp.int32),
        pltpu.VMEM((CHUNK, D), jnp.float32),
        pltpu.VMEM((CHUNK, D), jnp.float32),
        pltpu.VMEM_SHARED((NP, D), jnp.float32),
        pltpu.SemaphoreType.DMA,
        pltpu.SemaphoreType.DMA,
        pltpu.SemaphoreType.DMA,
        pltpu.SemaphoreType.DMA,
    ],
)
def _sc_agg(x_hbm, src_hbm, dstf_hbm, zf_hbm, parts_hbm,
            src_v, dst0, dst1, rows0, rows1, acc, sem0, sem1, semd0, semd1):
    cid = lax.axis_index("c")
    sid = lax.axis_index("s")
    wid = sid * NC + cid
    ebase = wid * (T_PER_W * CHUNK)

    pltpu.sync_copy(zf_hbm.at[pl.ds(sid * RPS_ACC, RPS_ACC)],
                    acc.at[pl.ds(sid * RPS_ACC, RPS_ACC)])
    plsc.subcore_barrier()

    @pl.loop(0, T_PER_W, step=KI)
    def _(t0):
        pltpu.sync_copy(src_hbm.at[wid, pl.ds(t0, KI)], src_v)

        @pl.loop(0, KI, step=2)
        def _(j):
            g0 = pltpu.async_copy(x_hbm.at[src_v.at[j]], rows0, sem0)
            g1 = pltpu.async_copy(x_hbm.at[src_v.at[j + 1]], rows1, sem1)
            d0 = pltpu.async_copy(
                dstf_hbm.at[pl.ds(ebase + (t0 + j) * CHUNK, CHUNK)],
                dst0, semd0)
            d1 = pltpu.async_copy(
                dstf_hbm.at[pl.ds(ebase + (t0 + j + 1) * CHUNK, CHUNK)],
                dst1, semd1)
            g0.wait()
            d0.wait()
            pltpu.sync_copy(rows0, acc.at[dst0], add=True)
            g1.wait()
            d1.wait()
            pltpu.sync_copy(rows1, acc.at[dst1], add=True)

    plsc.subcore_barrier()
    pltpu.sync_copy(acc.at[pl.ds(sid * RPS_OUT, RPS_OUT)],
                    parts_hbm.at[cid, pl.ds(sid * RPS_OUT, RPS_OUT)])

    @pl.when(sid == 0)
    def _():
        pltpu.sync_copy(acc.at[pl.ds(NS * RPS_OUT, TAIL)],
                        parts_hbm.at[cid, pl.ds(NS * RPS_OUT, TAIL)])


@functools.partial(
    pl.kernel,
    out_type=jax.ShapeDtypeStruct((NC, N, D), jnp.float32),
    mesh=_mesh,
    scratch_types=[
        pltpu.VMEM((CHUNK,), jnp.int32),
        pltpu.VMEM((CHUNK,), jnp.int32),
        pltpu.VMEM((CHUNK, D), jnp.float32),
        pltpu.VMEM_SHARED((NP, D), jnp.float32),
        pltpu.SemaphoreType.DMA,
        pltpu.SemaphoreType.DMA,
    ],
)
def _sc_counts(dstf_hbm, zc_hbm, ones_hbm, cnts_hbm, dst0, dst1, ones_v,
               cacc, semd0, semd1):
    cid = lax.axis_index("c")
    sid = lax.axis_index("s")
    wid = sid * NC + cid
    ebase = wid * (T_PER_W * CHUNK)

    pltpu.sync_copy(zc_hbm.at[pl.ds(sid * RPS_ACC, RPS_ACC)],
                    cacc.at[pl.ds(sid * RPS_ACC, RPS_ACC)])
    pltpu.sync_copy(ones_hbm, ones_v)
    plsc.subcore_barrier()

    @pl.loop(0, T_PER_W, step=2)
    def _(t):
        d0 = pltpu.async_copy(dstf_hbm.at[pl.ds(ebase + t * CHUNK, CHUNK)],
                              dst0, semd0)
        d1 = pltpu.async_copy(
            dstf_hbm.at[pl.ds(ebase + (t + 1) * CHUNK, CHUNK)], dst1, semd1)
        d0.wait()
        pltpu.sync_copy(ones_v, cacc.at[dst0], add=True)
        d1.wait()
        pltpu.sync_copy(ones_v, cacc.at[dst1], add=True)

    plsc.subcore_barrier()
    pltpu.sync_copy(cacc.at[pl.ds(sid * RPS_OUT, RPS_OUT)],
                    cnts_hbm.at[cid, pl.ds(sid * RPS_OUT, RPS_OUT)])

    @pl.when(sid == 0)
    def _():
        pltpu.sync_copy(cacc.at[pl.ds(NS * RPS_OUT, TAIL)],
                        cnts_hbm.at[cid, pl.ds(NS * RPS_OUT, TAIL)])


BR = 1000
_PREC = lax.Precision.HIGHEST


def _dense1_body(parts_ref, cnts_ref, x_ref, wl_ref, b_ref, wr_ref, o_ref):
    p = parts_ref[0] + parts_ref[1]
    c = cnts_ref[0, :, 0:1] + cnts_ref[1, :, 0:1]
    mean = p / jnp.maximum(c, 1.0)
    h = (jnp.dot(mean, wl_ref[...], precision=_PREC,
                 preferred_element_type=jnp.float32)
         + b_ref[...]
         + jnp.dot(x_ref[...], wr_ref[...], precision=_PREC,
                   preferred_element_type=jnp.float32))
    o_ref[...] = jnp.maximum(h, 0.0)


def _dense2_body(parts_ref, cnts_ref, h1_ref, x_ref, wl_ref, b_ref, wr_ref,
                 wo_ref, bo_ref, o_ref):
    p = parts_ref[0] + parts_ref[1]
    c = cnts_ref[0, :, 0:1] + cnts_ref[1, :, 0:1]
    mean = p / jnp.maximum(c, 1.0)
    h2 = (jnp.dot(mean, wl_ref[...], precision=_PREC,
                  preferred_element_type=jnp.float32)
          + b_ref[...]
          + jnp.dot(h1_ref[...], wr_ref[...], precision=_PREC,
                    preferred_element_type=jnp.float32))
    h2 = jnp.maximum(h2, 0.0)
    out = x_ref[...] + jnp.dot(h2, wo_ref[...], precision=_PREC,
                               preferred_element_type=jnp.float32) + bo_ref[...]
    o_ref[...] = jnp.maximum(out, 0.0)


_spec_parts = pl.BlockSpec((NC, BR, D), lambda i: (0, i, 0))
_spec_cnts = pl.BlockSpec((NC, BR, D), lambda i: (0, i, 0))
_spec_rows = pl.BlockSpec((BR, D), lambda i: (i, 0))
_spec_w = pl.BlockSpec((D, H), lambda i: (0, 0))
_spec_b = pl.BlockSpec((1, H), lambda i: (0, 0))

_dense1 = pl.pallas_call(
    _dense1_body,
    grid=(N // BR,),
    in_specs=[_spec_parts, _spec_cnts, _spec_rows, _spec_w, _spec_b, _spec_w],
    out_specs=_spec_rows,
    out_shape=jax.ShapeDtypeStruct((N, H), jnp.float32),
)

_dense2 = pl.pallas_call(
    _dense2_body,
    grid=(N // BR,),
    in_specs=[_spec_parts, _spec_cnts, _spec_rows, _spec_rows, _spec_w,
              _spec_b, _spec_w, _spec_w, _spec_b],
    out_specs=_spec_rows,
    out_shape=jax.ShapeDtypeStruct((N, D), jnp.float32),
)


@jax.jit
def kernel(x, edge_index, W1l, b1l, W1r, W2l, b2l, W2r, Wout, bout):
    src = edge_index[0].astype(jnp.int32)
    dst = edge_index[1].astype(jnp.int32)
    pad = E_PAD - E
    pad_iota = jnp.arange(pad, dtype=jnp.int32)
    src_p = jnp.concatenate([src, pad_iota % N])
    dst_p = jnp.concatenate([dst, N + (pad_iota % (NP - N))])
    src_r = src_p.reshape(NW, T_PER_W, CHUNK)
    zf = jnp.zeros((NP, D), jnp.float32)

    cnts = _sc_counts(dst_p, zf, jnp.ones((CHUNK, D), jnp.float32))
    parts1 = _sc_agg(x, src_r, dst_p, zf)
    h1 = _dense1(parts1, cnts, x, W1l.T, b1l.reshape(1, H), W1r.T)
    parts2 = _sc_agg(h1, src_r, dst_p, zf)
    out = _dense2(parts2, cnts, h1, x, W2l.T, b2l.reshape(1, H), W2r.T,
                  Wout.T, bout.reshape(1, D))
    return out


# speedup vs baseline: 9.7144x; 1.2495x over previous
import functools

import jax
import jax.numpy as jnp
from jax import lax
from jax.experimental import pallas as pl
from jax.experimental.pallas import tpu as pltpu
from jax.experimental.pallas import tpu_sc as plsc

N = 10000
D = 128
H = 128
E = 320000

NC = 2
NS = 16
NW = NC * NS
CHUNK = 64
T_PER_W = 160
KI = 32
NBUF = 4
CCHUNK = 128
CT_PER_W = 80
E_PAD = NW * T_PER_W * CHUNK
NP = 10112
RPS_ACC = NP // NS
RPS_OUT = 624
TAIL = N - NS * RPS_OUT

_mesh = plsc.VectorSubcoreMesh(core_axis_name="c", subcore_axis_name="s",
                               num_cores=NC, num_subcores=NS)


@functools.partial(
    pl.kernel,
    out_type=jax.ShapeDtypeStruct((NC, N, D), jnp.float32),
    mesh=_mesh,
    scratch_types=[
        pltpu.VMEM((KI, CHUNK), jnp.int32),
        [pltpu.VMEM((CHUNK,), jnp.int32) for _ in range(NBUF)],
        [pltpu.VMEM((CHUNK, D), jnp.float32) for _ in range(NBUF)],
        pltpu.VMEM_SHARED((NP, D), jnp.float32),
        [pltpu.SemaphoreType.DMA for _ in range(NBUF)],
        [pltpu.SemaphoreType.DMA for _ in range(NBUF)],
    ],
)
def _sc_agg(x_hbm, src_hbm, dstf_hbm, zf_hbm, parts_hbm,
            src_v, dsts, rows, acc, gsems, dsems):
    cid = lax.axis_index("c")
    sid = lax.axis_index("s")
    wid = sid * NC + cid
    ebase = wid * (T_PER_W * CHUNK)

    pltpu.sync_copy(zf_hbm.at[pl.ds(sid * RPS_ACC, RPS_ACC)],
                    acc.at[pl.ds(sid * RPS_ACC, RPS_ACC)])
    plsc.subcore_barrier()

    @pl.loop(0, T_PER_W, step=KI)
    def _(t0):
        pltpu.sync_copy(src_hbm.at[wid, pl.ds(t0, KI)], src_v)

        for b in range(NBUF):
            pltpu.async_copy(x_hbm.at[src_v.at[b]], rows[b], gsems[b])
            pltpu.async_copy(
                dstf_hbm.at[pl.ds(ebase + (t0 + b) * CHUNK, CHUNK)],
                dsts[b], dsems[b])

        @pl.loop(0, KI, step=NBUF)
        def _(j):
            for b in range(NBUF):
                pltpu.make_async_copy(x_hbm.at[src_v.at[j + b]], rows[b],
                                      gsems[b]).wait()
                pltpu.make_async_copy(
                    dstf_hbm.at[pl.ds(ebase + (t0 + j + b) * CHUNK, CHUNK)],
                    dsts[b], dsems[b]).wait()
                pltpu.sync_copy(rows[b], acc.at[dsts[b]], add=True)

                @pl.when(j + b + NBUF < KI)
                def _(b=b):
                    pltpu.async_copy(x_hbm.at[src_v.at[j + b + NBUF]],
                                     rows[b], gsems[b])
                    pltpu.async_copy(
                        dstf_hbm.at[pl.ds(
                            ebase + (t0 + j + b + NBUF) * CHUNK, CHUNK)],
                        dsts[b], dsems[b])

    plsc.subcore_barrier()
    pltpu.sync_copy(acc.at[pl.ds(sid * RPS_OUT, RPS_OUT)],
                    parts_hbm.at[cid, pl.ds(sid * RPS_OUT, RPS_OUT)])

    @pl.when(sid == 0)
    def _():
        pltpu.sync_copy(acc.at[pl.ds(NS * RPS_OUT, TAIL)],
                        parts_hbm.at[cid, pl.ds(NS * RPS_OUT, TAIL)])


@functools.partial(
    pl.kernel,
    out_type=jax.ShapeDtypeStruct((NC, N, D), jnp.float32),
    mesh=_mesh,
    scratch_types=[
        pltpu.VMEM((CCHUNK,), jnp.int32),
        pltpu.VMEM((CCHUNK,), jnp.int32),
        pltpu.VMEM((CCHUNK, D), jnp.float32),
        pltpu.VMEM_SHARED((NP, D), jnp.float32),
        pltpu.SemaphoreType.DMA,
        pltpu.SemaphoreType.DMA,
    ],
)
def _sc_counts(dstf_hbm, zc_hbm, ones_hbm, cnts_hbm, dst0, dst1, ones_v,
               cacc, semd0, semd1):
    cid = lax.axis_index("c")
    sid = lax.axis_index("s")
    wid = sid * NC + cid
    ebase = wid * (CT_PER_W * CCHUNK)

    pltpu.sync_copy(zc_hbm.at[pl.ds(sid * RPS_ACC, RPS_ACC)],
                    cacc.at[pl.ds(sid * RPS_ACC, RPS_ACC)])
    pltpu.sync_copy(ones_hbm, ones_v)
    plsc.subcore_barrier()

    @pl.loop(0, CT_PER_W, step=2)
    def _(t):
        d0 = pltpu.async_copy(dstf_hbm.at[pl.ds(ebase + t * CCHUNK, CCHUNK)],
                              dst0, semd0)
        d1 = pltpu.async_copy(
            dstf_hbm.at[pl.ds(ebase + (t + 1) * CCHUNK, CCHUNK)], dst1, semd1)
        d0.wait()
        pltpu.sync_copy(ones_v, cacc.at[dst0], add=True)
        d1.wait()
        pltpu.sync_copy(ones_v, cacc.at[dst1], add=True)

    plsc.subcore_barrier()
    pltpu.sync_copy(cacc.at[pl.ds(sid * RPS_OUT, RPS_OUT)],
                    cnts_hbm.at[cid, pl.ds(sid * RPS_OUT, RPS_OUT)])

    @pl.when(sid == 0)
    def _():
        pltpu.sync_copy(cacc.at[pl.ds(NS * RPS_OUT, TAIL)],
                        cnts_hbm.at[cid, pl.ds(NS * RPS_OUT, TAIL)])


BR = 1000
_PREC = lax.Precision.HIGHEST


def _dense1_body(parts_ref, cnts_ref, x_ref, wl_ref, b_ref, wr_ref, o_ref):
    p = parts_ref[0] + parts_ref[1]
    c = cnts_ref[0, :, 0:1] + cnts_ref[1, :, 0:1]
    mean = p / jnp.maximum(c, 1.0)
    h = (jnp.dot(mean, wl_ref[...], precision=_PREC,
                 preferred_element_type=jnp.float32)
         + b_ref[...]
         + jnp.dot(x_ref[...], wr_ref[...], precision=_PREC,
                   preferred_element_type=jnp.float32))
    o_ref[...] = jnp.maximum(h, 0.0)


def _dense2_body(parts_ref, cnts_ref, h1_ref, x_ref, wl_ref, b_ref, wr_ref,
                 wo_ref, bo_ref, o_ref):
    p = parts_ref[0] + parts_ref[1]
    c = cnts_ref[0, :, 0:1] + cnts_ref[1, :, 0:1]
    mean = p / jnp.maximum(c, 1.0)
    h2 = (jnp.dot(mean, wl_ref[...], precision=_PREC,
                  preferred_element_type=jnp.float32)
          + b_ref[...]
          + jnp.dot(h1_ref[...], wr_ref[...], precision=_PREC,
                    preferred_element_type=jnp.float32))
    h2 = jnp.maximum(h2, 0.0)
    out = x_ref[...] + jnp.dot(h2, wo_ref[...], precision=_PREC,
                               preferred_element_type=jnp.float32) + bo_ref[...]
    o_ref[...] = jnp.maximum(out, 0.0)


_spec_parts = pl.BlockSpec((NC, BR, D), lambda i: (0, i, 0))
_spec_cnts = pl.BlockSpec((NC, BR, D), lambda i: (0, i, 0))
_spec_rows = pl.BlockSpec((BR, D), lambda i: (i, 0))
_spec_w = pl.BlockSpec((D, H), lambda i: (0, 0))
_spec_b = pl.BlockSpec((1, H), lambda i: (0, 0))

_dense1 = pl.pallas_call(
    _dense1_body,
    grid=(N // BR,),
    in_specs=[_spec_parts, _spec_cnts, _spec_rows, _spec_w, _spec_b, _spec_w],
    out_specs=_spec_rows,
    out_shape=jax.ShapeDtypeStruct((N, H), jnp.float32),
)

_dense2 = pl.pallas_call(
    _dense2_body,
    grid=(N // BR,),
    in_specs=[_spec_parts, _spec_cnts, _spec_rows, _spec_rows, _spec_w,
              _spec_b, _spec_w, _spec_w, _spec_b],
    out_specs=_spec_rows,
    out_shape=jax.ShapeDtypeStruct((N, D), jnp.float32),
)


@jax.jit
def kernel(x, edge_index, W1l, b1l, W1r, W2l, b2l, W2r, Wout, bout):
    src = edge_index[0].astype(jnp.int32)
    dst = edge_index[1].astype(jnp.int32)
    pad = E_PAD - E
    pad_iota = jnp.arange(pad, dtype=jnp.int32)
    src_p = jnp.concatenate([src, pad_iota % N])
    dst_p = jnp.concatenate([dst, N + (pad_iota % (NP - N))])
    src_r = src_p.reshape(NW, T_PER_W, CHUNK)
    zf = jnp.zeros((NP, D), jnp.float32)

    cnts = _sc_counts(dst_p, zf, jnp.ones((CCHUNK, D), jnp.float32))
    parts1 = _sc_agg(x, src_r, dst_p, zf)
    h1 = _dense1(parts1, cnts, x, W1l.T, b1l.reshape(1, H), W1r.T)
    parts2 = _sc_agg(h1, src_r, dst_p, zf)
    out = _dense2(parts2, cnts, h1, x, W2l.T, b2l.reshape(1, H), W2r.T,
                  Wout.T, bout.reshape(1, D))
    return out


# speedup vs baseline: 10.0601x; 1.0356x over previous
import functools

import jax
import jax.numpy as jnp
from jax import lax
from jax.experimental import pallas as pl
from jax.experimental.pallas import tpu as pltpu
from jax.experimental.pallas import tpu_sc as plsc

N = 10000
D = 128
H = 128
E = 320000

NC = 2
NS = 16
NW = NC * NS
CHUNK = 64
T_PER_W = 160
KI = 80
NBUF = 4
CCHUNK = 128
CT_PER_W = 80
E_PAD = NW * T_PER_W * CHUNK
NP = 10112
RPS_ACC = NP // NS
RPS_OUT = 624
TAIL = N - NS * RPS_OUT

_mesh = plsc.VectorSubcoreMesh(core_axis_name="c", subcore_axis_name="s",
                               num_cores=NC, num_subcores=NS)


@functools.partial(
    pl.kernel,
    out_type=jax.ShapeDtypeStruct((NC, N, D), jnp.float32),
    mesh=_mesh,
    scratch_types=[
        pltpu.VMEM((KI, CHUNK), jnp.int32),
        [pltpu.VMEM((CHUNK,), jnp.int32) for _ in range(NBUF)],
        [pltpu.VMEM((CHUNK, D), jnp.float32) for _ in range(NBUF)],
        pltpu.VMEM_SHARED((NP, D), jnp.float32),
        [pltpu.SemaphoreType.DMA for _ in range(NBUF)],
        [pltpu.SemaphoreType.DMA for _ in range(NBUF)],
    ],
)
def _sc_agg(x_hbm, src_hbm, dstf_hbm, zf_hbm, parts_hbm,
            src_v, dsts, rows, acc, gsems, dsems):
    cid = lax.axis_index("c")
    sid = lax.axis_index("s")
    wid = sid * NC + cid
    ebase = wid * (T_PER_W * CHUNK)

    pltpu.sync_copy(zf_hbm.at[pl.ds(sid * RPS_ACC, RPS_ACC)],
                    acc.at[pl.ds(sid * RPS_ACC, RPS_ACC)])
    plsc.subcore_barrier()

    @pl.loop(0, T_PER_W, step=KI)
    def _(t0):
        pltpu.sync_copy(src_hbm.at[wid, pl.ds(t0, KI)], src_v)

        for b in range(NBUF):
            pltpu.async_copy(x_hbm.at[src_v.at[b]], rows[b], gsems[b])
            pltpu.async_copy(
                dstf_hbm.at[pl.ds(ebase + (t0 + b) * CHUNK, CHUNK)],
                dsts[b], dsems[b])

        @pl.loop(0, KI, step=NBUF)
        def _(j):
            for b in range(NBUF):
                pltpu.make_async_copy(x_hbm.at[src_v.at[j + b]], rows[b],
                                      gsems[b]).wait()
                pltpu.make_async_copy(
                    dstf_hbm.at[pl.ds(ebase + (t0 + j + b) * CHUNK, CHUNK)],
                    dsts[b], dsems[b]).wait()
                pltpu.sync_copy(rows[b], acc.at[dsts[b]], add=True)

                @pl.when(j + b + NBUF < KI)
                def _(b=b):
                    pltpu.async_copy(x_hbm.at[src_v.at[j + b + NBUF]],
                                     rows[b], gsems[b])
                    pltpu.async_copy(
                        dstf_hbm.at[pl.ds(
                            ebase + (t0 + j + b + NBUF) * CHUNK, CHUNK)],
                        dsts[b], dsems[b])

    plsc.subcore_barrier()
    pltpu.sync_copy(acc.at[pl.ds(sid * RPS_OUT, RPS_OUT)],
                    parts_hbm.at[cid, pl.ds(sid * RPS_OUT, RPS_OUT)])

    @pl.when(sid == 0)
    def _():
        pltpu.sync_copy(acc.at[pl.ds(NS * RPS_OUT, TAIL)],
                        parts_hbm.at[cid, pl.ds(NS * RPS_OUT, TAIL)])


@functools.partial(
    pl.kernel,
    out_type=jax.ShapeDtypeStruct((NC, N, D), jnp.float32),
    mesh=_mesh,
    scratch_types=[
        pltpu.VMEM((CCHUNK,), jnp.int32),
        pltpu.VMEM((CCHUNK,), jnp.int32),
        pltpu.VMEM((CCHUNK, D), jnp.float32),
        pltpu.VMEM_SHARED((NP, D), jnp.float32),
        pltpu.SemaphoreType.DMA,
        pltpu.SemaphoreType.DMA,
    ],
)
def _sc_counts(dstf_hbm, zc_hbm, ones_hbm, cnts_hbm, dst0, dst1, ones_v,
               cacc, semd0, semd1):
    cid = lax.axis_index("c")
    sid = lax.axis_index("s")
    wid = sid * NC + cid
    ebase = wid * (CT_PER_W * CCHUNK)

    pltpu.sync_copy(zc_hbm.at[pl.ds(sid * RPS_ACC, RPS_ACC)],
                    cacc.at[pl.ds(sid * RPS_ACC, RPS_ACC)])
    pltpu.sync_copy(ones_hbm, ones_v)
    plsc.subcore_barrier()

    @pl.loop(0, CT_PER_W, step=2)
    def _(t):
        d0 = pltpu.async_copy(dstf_hbm.at[pl.ds(ebase + t * CCHUNK, CCHUNK)],
                              dst0, semd0)
        d1 = pltpu.async_copy(
            dstf_hbm.at[pl.ds(ebase + (t + 1) * CCHUNK, CCHUNK)], dst1, semd1)
        d0.wait()
        pltpu.sync_copy(ones_v, cacc.at[dst0], add=True)
        d1.wait()
        pltpu.sync_copy(ones_v, cacc.at[dst1], add=True)

    plsc.subcore_barrier()
    pltpu.sync_copy(cacc.at[pl.ds(sid * RPS_OUT, RPS_OUT)],
                    cnts_hbm.at[cid, pl.ds(sid * RPS_OUT, RPS_OUT)])

    @pl.when(sid == 0)
    def _():
        pltpu.sync_copy(cacc.at[pl.ds(NS * RPS_OUT, TAIL)],
                        cnts_hbm.at[cid, pl.ds(NS * RPS_OUT, TAIL)])


BR = 1000
_PREC = lax.Precision.HIGHEST


def _dense1_body(parts_ref, cnts_ref, x_ref, wl_ref, b_ref, wr_ref, o_ref):
    p = parts_ref[0] + parts_ref[1]
    c = cnts_ref[0, :, 0:1] + cnts_ref[1, :, 0:1]
    mean = p / jnp.maximum(c, 1.0)
    h = (jnp.dot(mean, wl_ref[...], precision=_PREC,
                 preferred_element_type=jnp.float32)
         + b_ref[...]
         + jnp.dot(x_ref[...], wr_ref[...], precision=_PREC,
                   preferred_element_type=jnp.float32))
    o_ref[...] = jnp.maximum(h, 0.0)


def _dense2_body(parts_ref, cnts_ref, h1_ref, x_ref, wl_ref, b_ref, wr_ref,
                 wo_ref, bo_ref, o_ref):
    p = parts_ref[0] + parts_ref[1]
    c = cnts_ref[0, :, 0:1] + cnts_ref[1, :, 0:1]
    mean = p / jnp.maximum(c, 1.0)
    h2 = (jnp.dot(mean, wl_ref[...], precision=_PREC,
                  preferred_element_type=jnp.float32)
          + b_ref[...]
          + jnp.dot(h1_ref[...], wr_ref[...], precision=_PREC,
                    preferred_element_type=jnp.float32))
    h2 = jnp.maximum(h2, 0.0)
    out = x_ref[...] + jnp.dot(h2, wo_ref[...], precision=_PREC,
                               preferred_element_type=jnp.float32) + bo_ref[...]
    o_ref[...] = jnp.maximum(out, 0.0)


_spec_parts = pl.BlockSpec((NC, BR, D), lambda i: (0, i, 0))
_spec_cnts = pl.BlockSpec((NC, BR, D), lambda i: (0, i, 0))
_spec_rows = pl.BlockSpec((BR, D), lambda i: (i, 0))
_spec_w = pl.BlockSpec((D, H), lambda i: (0, 0))
_spec_b = pl.BlockSpec((1, H), lambda i: (0, 0))

_dense1 = pl.pallas_call(
    _dense1_body,
    grid=(N // BR,),
    in_specs=[_spec_parts, _spec_cnts, _spec_rows, _spec_w, _spec_b, _spec_w],
    out_specs=_spec_rows,
    out_shape=jax.ShapeDtypeStruct((N, H), jnp.float32),
)

_dense2 = pl.pallas_call(
    _dense2_body,
    grid=(N // BR,),
    in_specs=[_spec_parts, _spec_cnts, _spec_rows, _spec_rows, _spec_w,
              _spec_b, _spec_w, _spec_w, _spec_b],
    out_specs=_spec_rows,
    out_shape=jax.ShapeDtypeStruct((N, D), jnp.float32),
)


@jax.jit
def kernel(x, edge_index, W1l, b1l, W1r, W2l, b2l, W2r, Wout, bout):
    src = edge_index[0].astype(jnp.int32)
    dst = edge_index[1].astype(jnp.int32)
    pad = E_PAD - E
    pad_iota = jnp.arange(pad, dtype=jnp.int32)
    src_p = jnp.concatenate([src, pad_iota % N])
    dst_p = jnp.concatenate([dst, N + (pad_iota % (NP - N))])
    src_r = src_p.reshape(NW, T_PER_W, CHUNK)
    zf = jnp.zeros((NP, D), jnp.float32)

    cnts = _sc_counts(dst_p, zf, jnp.ones((CCHUNK, D), jnp.float32))
    parts1 = _sc_agg(x, src_r, dst_p, zf)
    h1 = _dense1(parts1, cnts, x, W1l.T, b1l.reshape(1, H), W1r.T)
    parts2 = _sc_agg(h1, src_r, dst_p, zf)
    out = _dense2(parts2, cnts, h1, x, W2l.T, b2l.reshape(1, H), W2r.T,
                  Wout.T, bout.reshape(1, D))
    return out
